# baseline (speedup 1.0000x reference)
import functools

import jax
import jax.numpy as jnp
from jax import lax
from jax.experimental import pallas as pl
from jax.experimental.pallas import tpu as pltpu
from jax.experimental.pallas import tpu_sc as plsc

BATCH = 16384
DIM = 64
LANES = 16
NUM_CORES = 2
NUM_SUBCORES = 16
NUM_WORKERS = NUM_CORES * NUM_SUBCORES
BPW = BATCH // NUM_WORKERS
GROUPS = BPW // LANES


def _mf_body(uidx_hbm, iidx_hbm, uemb_hbm, iemb_hbm, ubias_hbm, ibias_hbm,
             gbias_hbm, out_hbm,
             uidx_v, iidx_v, urows_v, irows_v, ubias_v, ibias_v, gb_v,
             out_v, sem):
    wid = lax.axis_index("s") * NUM_CORES + lax.axis_index("c")
    base = wid * BPW

    pltpu.sync_copy(uidx_hbm.at[pl.ds(base, BPW)], uidx_v)
    pltpu.sync_copy(iidx_hbm.at[pl.ds(base, BPW)], iidx_v)

    c1 = pltpu.async_copy(uemb_hbm.at[uidx_v], urows_v, sem)
    c2 = pltpu.async_copy(iemb_hbm.at[iidx_v], irows_v, sem)
    c3 = pltpu.async_copy(ubias_hbm.at[uidx_v], ubias_v, sem)
    c4 = pltpu.async_copy(ibias_hbm.at[iidx_v], ibias_v, sem)
    pltpu.sync_copy(gbias_hbm, gb_v)
    c1.wait()
    c2.wait()
    c3.wait()
    c4.wait()

    gb = gb_v[pl.ds(0, LANES)]

    def group(g, carry):
        row_ids = g * LANES + lax.iota(jnp.int32, LANES)
        r0 = g * LANES
        acc = gb + ubias_v[pl.ds(r0, LANES)] + ibias_v[pl.ds(r0, LANES)]
        for d in range(DIM):
            d_vec = jnp.full((LANES,), d, jnp.int32)
            u_col = plsc.load_gather(urows_v, [row_ids, d_vec])
            i_col = plsc.load_gather(irows_v, [row_ids, d_vec])
            acc = acc + u_col * i_col
        out_v[pl.ds(r0, LANES)] = acc
        return carry

    lax.fori_loop(0, GROUPS, group, 0, unroll=2)

    pltpu.sync_copy(out_v, out_hbm.at[pl.ds(base, BPW)])


@jax.jit
def kernel(user_indices, item_indices, user_embedding, item_embedding,
           user_bias, item_bias, global_bias):
    mesh = plsc.VectorSubcoreMesh(core_axis_name="c", subcore_axis_name="s")
    run = functools.partial(
        pl.kernel,
        mesh=mesh,
        compiler_params=pltpu.CompilerParams(needs_layout_passes=False,
                                             use_tc_tiling_on_sc=False),
        out_type=jax.ShapeDtypeStruct((BATCH,), jnp.float32),
        scratch_types=[
            pltpu.VMEM((BPW,), jnp.int32),
            pltpu.VMEM((BPW,), jnp.int32),
            pltpu.VMEM((BPW, DIM), jnp.float32),
            pltpu.VMEM((BPW, DIM), jnp.float32),
            pltpu.VMEM((BPW,), jnp.float32),
            pltpu.VMEM((BPW,), jnp.float32),
            pltpu.VMEM((LANES,), jnp.float32),
            pltpu.VMEM((BPW,), jnp.float32),
            pltpu.SemaphoreType.DMA,
        ],
    )(_mf_body)
    return run(user_indices.astype(jnp.int32), item_indices.astype(jnp.int32),
               user_embedding, item_embedding,
               user_bias.reshape(-1), item_bias.reshape(-1),
               jnp.broadcast_to(global_bias, (LANES,)))
